# baseline (speedup 1.0000x reference)
import functools

import jax
import jax.numpy as jnp
from jax import lax
from jax.experimental import pallas as pl
from jax.experimental.pallas import tpu as pltpu
from jax.experimental.pallas import tpu_sc as plsc

NCLS = 35
KP = 64
LAM = 0.2
D = 512
N = 16384

N_SC = 4096
N_TC = N - N_SC
R = 2048
G = N_TC // R

NW = 32
RPW = N_SC // NW
CH = 64
NCH = RPW // CH
AC = 640


def _tc_body(x_ref, lab_ref, sums_ref, aux_ref):
    i = pl.program_id(0)

    @pl.when(i == 0)
    def _init():
        sums_ref[...] = jnp.zeros_like(sums_ref)
        aux_ref[...] = jnp.zeros_like(aux_ref)

    x = x_ref[...]
    r = jnp.sum(x * x, axis=1, keepdims=True)
    scale = 1.0 / jnp.maximum(jnp.sqrt(r), 1e-12)
    lab = lab_ref[0, 0, :]
    iota = lax.broadcasted_iota(jnp.int32, (R, KP), 1)
    onehot = (lab[:, None] == iota).astype(jnp.float32)
    oh_scaled = (onehot * scale).astype(jnp.bfloat16)
    sums_ref[...] += lax.dot_general(
        oh_scaled, x.astype(jnp.bfloat16), (((0,), (0,)), ((), ())),
        preferred_element_type=jnp.float32)
    iota2 = lax.broadcasted_iota(jnp.int32, (R, 128), 1)
    v2 = jnp.where(iota2 == 0, 1.0, jnp.where(iota2 == 1, r * scale * scale, 0.0))
    aux_ref[...] += lax.dot_general(
        onehot, v2, (((0,), (0,)), ((), ())),
        preferred_element_type=jnp.float32)


def _tc_partials(x_tc, lab3):
    return pl.pallas_call(
        _tc_body,
        grid=(G,),
        in_specs=[
            pl.BlockSpec((R, D), lambda i: (i, 0)),
            pl.BlockSpec((1, 1, R), lambda i: (i, 0, 0)),
        ],
        out_specs=[
            pl.BlockSpec((KP, D), lambda i: (0, 0)),
            pl.BlockSpec((KP, 128), lambda i: (0, 0)),
        ],
        out_shape=[
            jax.ShapeDtypeStruct((KP, D), jnp.float32),
            jax.ShapeDtypeStruct((KP, 128), jnp.float32),
        ],
    )(x_tc, lab3)


def _sc_body(x_hbm, lab_hbm, out_hbm, rows_v, lab_v, acc_v):
    c = lax.axis_index("c")
    s = lax.axis_index("s")
    wid = s * 2 + c
    base = N_TC + wid * RPW
    pltpu.sync_copy(lab_hbm.at[pl.ds(base, RPW)], lab_v.at[pl.ds(0, RPW)])

    zeros16 = jnp.zeros((16,), jnp.float32)

    def zero_row(k, carry):
        for cc in range(AC // 16):
            acc_v[k, pl.ds(cc * 16, 16)] = zeros16
        return carry

    lax.fori_loop(0, NCLS, zero_row, 0)

    i16 = lax.iota(jnp.int32, 16)

    def do_chunk(ci, carry):
        pltpu.sync_copy(x_hbm.at[pl.ds(base + ci * CH, CH)], rows_v)

        def do_row(j, inner):
            racc0 = zeros16
            racc1 = zeros16
            racc2 = zeros16
            racc3 = zeros16
            for cc in range(0, D // 16, 4):
                xv0 = rows_v[j, pl.ds(cc * 16, 16)]
                xv1 = rows_v[j, pl.ds((cc + 1) * 16, 16)]
                xv2 = rows_v[j, pl.ds((cc + 2) * 16, 16)]
                xv3 = rows_v[j, pl.ds((cc + 3) * 16, 16)]
                racc0 = racc0 + xv0 * xv0
                racc1 = racc1 + xv1 * xv1
                racc2 = racc2 + xv2 * xv2
                racc3 = racc3 + xv3 * xv3
            rv = jnp.full((16,), jnp.sum((racc0 + racc1) + (racc2 + racc3)))
            ii = 0x5F3759DF - (plsc.bitcast(rv, jnp.int32) >> 1)
            y = plsc.bitcast(ii, jnp.float32)
            y = y * (1.5 - 0.5 * rv * y * y)
            y = y * (1.5 - 0.5 * rv * y * y)
            y = y * (1.5 - 0.5 * rv * y * y)
            scale = jnp.where(rv < 1e-24, 1e12, y)
            lab = lab_v[pl.ds(ci * CH + j, 16)][0]
            for cc in range(D // 16):
                xv = rows_v[j, pl.ds(cc * 16, 16)]
                plsc.addupdate(acc_v.at[lab, pl.ds(cc * 16, 16)], xv * scale)
            s2v = rv * scale * scale
            extra = jnp.where(i16 == 0, 1.0, jnp.where(i16 == 1, s2v, 0.0))
            plsc.addupdate(acc_v.at[lab, pl.ds(D, 16)], extra)
            return inner

        lax.fori_loop(0, CH, do_row, 0, unroll=2)
        return carry

    lax.fori_loop(0, NCH, do_chunk, 0)
    pltpu.sync_copy(acc_v, out_hbm.at[wid])


_sc_stats = functools.partial(
    pl.kernel,
    mesh=plsc.VectorSubcoreMesh(core_axis_name="c", subcore_axis_name="s"),
    compiler_params=pltpu.CompilerParams(needs_layout_passes=False),
    out_type=jax.ShapeDtypeStruct((NW, NCLS, AC), jnp.float32),
    scratch_types=[
        pltpu.VMEM((CH, D), jnp.float32),
        pltpu.VMEM((RPW + 16,), jnp.int32),
        pltpu.VMEM((NCLS, AC), jnp.float32),
    ],
)(_sc_body)


def _fin_body(tsums_ref, taux_ref, p_ref, out_ref):
    sc = jnp.sum(p_ref[...], axis=0)
    sums = tsums_ref[0:NCLS, :] + sc[:, 0:D]
    cnt = taux_ref[0:NCLS, 0:1] + sc[:, D:D + 1]
    s2 = taux_ref[0:NCLS, 1:2] + sc[:, D + 1:D + 2]
    ssq = jnp.sum(sums * sums, axis=1, keepdims=True)
    safe = jnp.maximum(cnt, 1.0)
    contrib = jnp.where(cnt > 0.0, (s2 - ssq / safe) / safe, 0.0)
    out_ref[...] = LAM * jnp.sum(contrib, keepdims=True)


@jax.jit
def kernel(input, label):
    lab3 = label[:N_TC].reshape(G, 1, R)
    tsums, taux = _tc_partials(input, lab3)
    partials = _sc_stats(input, label)
    out = pl.pallas_call(
        _fin_body,
        in_specs=[
            pl.BlockSpec((KP, D), lambda: (0, 0)),
            pl.BlockSpec((KP, 128), lambda: (0, 0)),
            pl.BlockSpec((NW, NCLS, AC), lambda: (0, 0, 0)),
        ],
        out_specs=pl.BlockSpec((1, 1), lambda: (0, 0)),
        out_shape=jax.ShapeDtypeStruct((1, 1), jnp.float32),
    )(tsums, taux, partials)
    return out[0, 0]


# speedup vs baseline: 3.2926x; 3.2926x over previous
import functools

import jax
import jax.numpy as jnp
from jax.experimental import pallas as pl
from jax.experimental.pallas import tpu as pltpu

NCLS = 35
KP = 64
LAM = 0.2
D = 512
N = 16384
R = 4096
G = N // R


def _body(x_ref, lab_ref, out_ref, sums_ref, cnt_ref, s2_ref):
    i = pl.program_id(0)

    @pl.when(i == 0)
    def _init():
        sums_ref[...] = jnp.zeros_like(sums_ref)
        cnt_ref[...] = jnp.zeros_like(cnt_ref)
        s2_ref[...] = jnp.zeros_like(s2_ref)

    x = x_ref[...]
    r = jnp.sum(x * x, axis=1, keepdims=True)
    scale = 1.0 / jnp.maximum(jnp.sqrt(r), 1e-12)
    lab = lab_ref[0, 0, :]
    iota = jax.lax.broadcasted_iota(jnp.int32, (R, KP), 1)
    onehot = (lab[:, None] == iota).astype(jnp.float32)
    oh_scaled = (onehot * scale).astype(jnp.bfloat16)
    sums_ref[...] += jax.lax.dot_general(
        x.astype(jnp.bfloat16), oh_scaled, (((0,), (0,)), ((), ())),
        preferred_element_type=jnp.float32)
    cnt_ref[...] += jnp.sum(onehot, axis=0, keepdims=True)
    s2_ref[...] += jnp.sum(onehot * (r * scale * scale), axis=0,
                           keepdims=True)

    @pl.when(i == G - 1)
    def _epilogue():
        sums = sums_ref[...]
        ssq = jnp.sum(sums * sums, axis=0, keepdims=True)
        cnt = cnt_ref[...]
        safe = jnp.maximum(cnt, 1.0)
        contrib = jnp.where(cnt > 0.0,
                            (s2_ref[...] - ssq / safe) / safe,
                            0.0)
        out_ref[...] = LAM * jnp.sum(contrib, keepdims=True)


@jax.jit
def kernel(input, label):
    lab3 = label.reshape(G, 1, R)
    out = pl.pallas_call(
        _body,
        grid=(G,),
        in_specs=[
            pl.BlockSpec((R, D), lambda i: (i, 0)),
            pl.BlockSpec((1, 1, R), lambda i: (i, 0, 0)),
        ],
        out_specs=pl.BlockSpec((1, 1), lambda i: (0, 0)),
        out_shape=jax.ShapeDtypeStruct((1, 1), jnp.float32),
        compiler_params=pltpu.CompilerParams(
            dimension_semantics=("arbitrary",)),
        scratch_shapes=[
            pltpu.VMEM((D, KP), jnp.float32),
            pltpu.VMEM((1, KP), jnp.float32),
            pltpu.VMEM((1, KP), jnp.float32),
        ],
    )(input, lab3)
    return out[0, 0]
